# baseline (speedup 1.0000x reference)
import functools
import math

import jax
import jax.numpy as jnp
from jax import lax
from jax.experimental import pallas as pl
from jax.experimental.pallas import tpu as pltpu
from jax.experimental.pallas import tpu_sc as plsc

D_MODEL = 64
SCALE = math.sqrt(D_MODEL)

_NC = 2
_NS = 16
_NW = _NC * _NS

_V = 1000002

_RCH = 96
_TPT = 31248
_PCH = 328

_ROWS = 4096
_COLS = 200
_XPT = _ROWS // _NW
_CH0, _CH1 = 96, 104
_NSTEP = _XPT * 2
_NBUF = 4
_PRE = 2


def _wid():
    return lax.axis_index("s") * _NC + lax.axis_index("c")


def _pack_body(table_hbm, lin_hbm, ibufs, pbufs, rsems, wsems):
    w = _wid()
    base = jnp.minimum(w * _TPT, _V - 2 - _RCH * _PCH)

    def row0(g):
        return jnp.minimum(base + g * _RCH, _V - 2 - _RCH)

    def read(g, b):
        return pltpu.make_async_copy(
            table_hbm.at[pl.ds(row0(g), _RCH), :], ibufs[b], rsems[b])

    def write(g, b):
        return pltpu.make_async_copy(
            pbufs[b], lin_hbm.at[pl.ds(row0(g), _RCH), :], wsems[b])

    def pack_scale(ib, pb, nrows, unroll):
        @plsc.parallel_loop(0, nrows, step=1, unroll=unroll)
        def _(r):
            for c in range(D_MODEL // 16):
                sl = pl.ds(c * 16, 16)
                sh = pl.ds(D_MODEL + c * 16, 16)
                v = ib[r, sl] * SCALE
                pb[r, sl] = v
                pb[r, sh] = v

    for s in range(_PRE):
        read(s, s % _NBUF).start()

    def group(grp, carry):
        for k in range(_NBUF):
            s = grp * _NBUF + k
            s_pre = s + _PRE
            k_pre = (k + _PRE) % _NBUF

            @pl.when(s_pre < _PCH)
            def _():
                @pl.when(s_pre >= _NBUF)
                def _():
                    write(s_pre - _NBUF, k_pre).wait()
                read(s_pre, k_pre).start()

            read(s, k).wait()
            pack_scale(ibufs[k], pbufs[k], _RCH, 4)
            write(s, k).start()
        return carry

    lax.fori_loop(0, _PCH // _NBUF, group, 0)

    for k in range(_NBUF):
        write(_PCH - _NBUF + k, k).wait()


@functools.partial(
    pl.kernel,
    out_type=jax.ShapeDtypeStruct((_V, 128), jnp.float32),
    mesh=plsc.VectorSubcoreMesh(core_axis_name="c", subcore_axis_name="s"),
    compiler_params=pltpu.CompilerParams(use_tc_tiling_on_sc=True),
    scratch_types=[
        [pltpu.VMEM((_RCH, D_MODEL), jnp.float32) for _ in range(_NBUF)],
        [pltpu.VMEM((_RCH, 128), jnp.float32) for _ in range(_NBUF)],
        [pltpu.SemaphoreType.DMA for _ in range(_NBUF)],
        [pltpu.SemaphoreType.DMA for _ in range(_NBUF)],
    ],
)
def _pack(table_hbm, lin_hbm, ibufs, pbufs, rsems, wsems):
    _pack_body(table_hbm, lin_hbm, ibufs, pbufs, rsems, wsems)


def _lookup_body(x_hbm, lin_hbm, out_hbm,
                 idxs, rows, obufs, isems, gsems, osems):
    w = _wid()
    i0 = w * _XPT
    flat0 = w * _XPT * _COLS

    def chunk_params(s, k):
        il = s // 2
        if k % 2 == 0:
            return il, 0, _CH0
        return il, _CH0, _CH1

    def idx_fetch(s, k):
        il, j0, ln = chunk_params(s, k)
        return pltpu.make_async_copy(
            x_hbm.at[pl.ds(flat0 + il * _COLS + j0, ln)], idxs[k], isems[k])

    def gather(s, k):
        _, _, ln = chunk_params(s, k)
        return pltpu.make_async_copy(
            lin_hbm.at[idxs[k]], rows[k].at[pl.ds(0, ln), :], gsems[k])

    def scatter(s, k):
        il, j0, ln = chunk_params(s, k)
        return pltpu.make_async_copy(
            obufs[k % 2].at[pl.ds(0, ln), :],
            out_hbm.at[i0 + il, pl.ds(j0, ln), :], osems[k % 2])

    def select(s, k):
        _, _, ln = chunk_params(s, k)
        rb = rows[k]
        ob = obufs[k % 2]

        @plsc.parallel_loop(0, ln, step=1, unroll=4)
        def _(j):
            for c in range(D_MODEL // 16):
                sl = pl.ds(c * 16, 16)
                ob[j, sl] = rb[j, sl]

    for s in range(_PRE + 1):
        idx_fetch(s, s % _NBUF).start()
    for s in range(_PRE):
        idx_fetch(s, s % _NBUF).wait()
        gather(s, s % _NBUF).start()

    def group(grp, carry):
        for k in range(_NBUF):
            s = grp * _NBUF + k
            s_pre = s + _PRE
            k_pre = (k + _PRE) % _NBUF
            s_ipre = s + _PRE + 1
            k_ipre = (k + _PRE + 1) % _NBUF

            @pl.when(s_ipre < _NSTEP)
            def _():
                idx_fetch(s_ipre, k_ipre).start()

            @pl.when(s_pre < _NSTEP)
            def _():
                idx_fetch(s_pre, k_pre).wait()
                gather(s_pre, k_pre).start()

            @pl.when(s >= 2)
            def _():
                scatter(s - 2, (k + 2) % _NBUF).wait()

            gather(s, k).wait()
            select(s, k)
            scatter(s, k).start()
        return carry

    lax.fori_loop(0, _NSTEP // _NBUF, group, 0)

    scatter(_NSTEP - 2, (_NSTEP - 2) % _NBUF).wait()
    scatter(_NSTEP - 1, (_NSTEP - 1) % _NBUF).wait()


@functools.partial(
    pl.kernel,
    out_type=jax.ShapeDtypeStruct((_ROWS, _COLS, D_MODEL), jnp.float32),
    mesh=plsc.VectorSubcoreMesh(core_axis_name="c", subcore_axis_name="s"),
    compiler_params=pltpu.CompilerParams(use_tc_tiling_on_sc=True),
    scratch_types=[
        [pltpu.VMEM((_CH0 if k % 2 == 0 else _CH1,), jnp.int32)
         for k in range(_NBUF)],
        [pltpu.VMEM((_CH1, 128), jnp.float32) for _ in range(_NBUF)],
        [pltpu.VMEM((_CH1, D_MODEL), jnp.float32) for _ in range(2)],
        [pltpu.SemaphoreType.DMA for _ in range(_NBUF)],
        [pltpu.SemaphoreType.DMA for _ in range(_NBUF)],
        [pltpu.SemaphoreType.DMA for _ in range(2)],
    ],
)
def _lookup(x_hbm, lin_hbm, out_hbm,
            idxs, rows, obufs, isems, gsems, osems):
    _lookup_body(x_hbm, lin_hbm, out_hbm,
                 idxs, rows, obufs, isems, gsems, osems)


@jax.jit
def kernel(x, table):
    lin = _pack(table)
    patch = table[_V - 2:, :] * SCALE
    lin = lax.dynamic_update_slice(
        lin, jnp.concatenate([patch, patch], axis=1), (_V - 2, 0))
    return _lookup(x.reshape(-1), lin)


# speedup vs baseline: 1.0323x; 1.0323x over previous
import functools
import math

import jax
import jax.numpy as jnp
from jax import lax
from jax.experimental import pallas as pl
from jax.experimental.pallas import tpu as pltpu
from jax.experimental.pallas import tpu_sc as plsc

D_MODEL = 64
SCALE = math.sqrt(D_MODEL)

_NC = 2
_NS = 16
_NW = _NC * _NS

_B = 4096 * 200
_B_W = _B // _NW
_CH = 128
_NSTEP = _B_W // _CH
_NBUF = 4
_PRE = 2
_NGRP = _NSTEP // _NBUF


def _body(x_hbm, table_hbm, out_hbm, idx_v, rows, gsems, osems, base):
    def gather(g, b):
        return pltpu.make_async_copy(
            table_hbm.at[idx_v.at[g]], rows[b], gsems[b])

    def scatter(g, b):
        return pltpu.make_async_copy(
            rows[b], out_hbm.at[pl.ds(base + g * _CH, _CH)], osems[b])

    pltpu.sync_copy(x_hbm, idx_v)

    for s in range(_PRE):
        gather(s, s % _NBUF).start()

    def group(grp, carry):
        for b in range(_NBUF):
            s = grp * _NBUF + b
            s_pre = s + _PRE
            b_pre = (b + _PRE) % _NBUF

            @pl.when(s_pre < _NSTEP)
            def _():
                @pl.when(s_pre >= _NBUF)
                def _():
                    scatter(s_pre - _NBUF, b_pre).wait()
                gather(s_pre, b_pre).start()

            gather(s, b).wait()

            buf = rows[b]

            @plsc.parallel_loop(0, _CH, step=1, unroll=8)
            def _scale(r):
                for c in range(D_MODEL // 16):
                    sl = pl.ds(c * 16, 16)
                    buf[r, sl] = buf[r, sl] * SCALE

            scatter(s, b).start()
        return carry

    lax.fori_loop(0, _NGRP, group, 0)

    for b in range(_NBUF):
        scatter(_NSTEP - _NBUF + b, b).wait()


@functools.partial(
    pl.kernel,
    out_type=jax.ShapeDtypeStruct((_B, D_MODEL), jnp.float32),
    mesh=plsc.VectorSubcoreMesh(core_axis_name="c", subcore_axis_name="s"),
    compiler_params=pltpu.CompilerParams(use_tc_tiling_on_sc=False),
    scratch_types=[
        pltpu.VMEM((_NSTEP, _CH), jnp.int32),
        [pltpu.VMEM((_CH, D_MODEL), jnp.float32) for _ in range(_NBUF)],
        [pltpu.SemaphoreType.DMA for _ in range(_NBUF)],
        [pltpu.SemaphoreType.DMA for _ in range(_NBUF)],
    ],
)
def _emb_lookup(x_hbm, table_hbm, out_hbm, idx_v, rows, gsems, osems):
    wid = lax.axis_index("s") * _NC + lax.axis_index("c")
    _body(x_hbm.at[wid], table_hbm, out_hbm, idx_v, rows,
          gsems, osems, wid * _B_W)


@jax.jit
def kernel(x, table):
    xg = x.reshape(_NW, _NSTEP, _CH)
    out = _emb_lookup(xg, table)
    return out.reshape(x.shape[0], x.shape[1], D_MODEL)
